# baseline (speedup 1.0000x reference)
import functools

import jax
import jax.numpy as jnp
from jax import lax
from jax.experimental import pallas as pl
from jax.experimental.pallas import tpu as pltpu
from jax.experimental.pallas import tpu_sc as plsc

N_NODES_C = 10000
N_EDGES_C = 320000
D_C = 128

NC = 2
NS = 16
LANES = 16

CH = 128
NCHT = 78
NXTRA = 4
DUMMY = N_NODES_C
ACC_PER_TILE = 632
ACC_ROWS = NS * ACC_PER_TILE


def _sc_body(feat_hbm, ei_hbm, out_hbm, *refs):
    idx = refs[0:3]
    rows = refs[3:6]
    acc = refs[6]
    isems = refs[7:10]
    gsems = refs[10:13]
    ssems = refs[13:16]

    c = lax.axis_index("c")
    s = lax.axis_index("s")
    w = c * NS + s

    def idx_start(b, eoff):
        pltpu.async_copy(
            ei_hbm.at[pl.ds(0, 2), pl.ds(eoff, CH)], idx[b], isems[b])

    def idx_wait(b):
        pltpu.make_async_copy(
            ei_hbm.at[pl.ds(0, 2), pl.ds(0, CH)], idx[b], isems[b]).wait()

    def mask(b):
        for j in range(CH // LANES):
            sl = pl.ds(j * LANES, LANES)
            sv = idx[b][0, sl]
            dv = idx[b][1, sl]
            idx[b][1, sl] = jnp.where(
                sv == dv, jnp.full((LANES,), DUMMY, jnp.int32), dv)

    H = CH // 2

    def g_start(b):
        pltpu.async_copy(feat_hbm.at[idx[b].at[0, pl.ds(0, H)]],
                         rows[b].at[pl.ds(0, H)], gsems[b])
        pltpu.async_copy(feat_hbm.at[idx[b].at[0, pl.ds(H, H)]],
                         rows[b].at[pl.ds(H, H)], gsems[b])

    def g_wait(b):
        pltpu.make_async_copy(feat_hbm.at[idx[b].at[0, pl.ds(0, H)]],
                              rows[b].at[pl.ds(0, H)], gsems[b]).wait()
        pltpu.make_async_copy(feat_hbm.at[idx[b].at[0, pl.ds(H, H)]],
                              rows[b].at[pl.ds(H, H)], gsems[b]).wait()

    def s_start(b):
        pltpu.async_copy(rows[b], acc.at[idx[b].at[1]], ssems[b], add=True)

    def s_wait(b):
        pltpu.make_async_copy(rows[b], acc.at[idx[b].at[1]], ssems[b]).wait()

    @pl.loop(0, CH)
    def _(i):
        @pl.loop(0, D_C // LANES)
        def _(k):
            rows[0][i, pl.ds(k * LANES, LANES)] = jnp.zeros(
                (LANES,), jnp.float32)

    abase = s * ACC_PER_TILE

    @pl.loop(0, 4)
    def _(j):
        pltpu.sync_copy(rows[0], acc.at[pl.ds(abase + j * CH, CH)])
    pltpu.sync_copy(rows[0].at[pl.ds(0, ACC_PER_TILE - 4 * CH)],
                    acc.at[pl.ds(abase + 4 * CH, ACC_PER_TILE - 4 * CH)])

    plsc.subcore_barrier()

    ebase = w * (NCHT * CH)
    xoff = (NS * NC * NCHT + 2 * c + s) * CH

    def eoff_of(kk, static_k, n):
        if static_k == n - 1 and n == NCHT + 1:
            return xoff
        return ebase + kk * CH

    def do_iter(k, b, n, k_traced=None):
        kk = k if k_traced is None else k_traced
        b1, b2 = (b + 1) % 3, (b + 2) % 3
        if k + 1 < n:
            idx_wait(b1)
            mask(b1)
            g_start(b1)
        g_wait(b)
        s_start(b)
        if k >= 1:
            s_wait(b2)
        if k + 2 < n:
            idx_start(b2, eoff_of(kk + 2, k + 2, n))

    def pipeline(n):
        idx_start(0, eoff_of(0, 0, n))
        idx_start(1, eoff_of(1, 1, n))
        idx_wait(0)
        mask(0)
        g_start(0)

        nmain = ((n - 6) // 3) * 3

        for k in range(0, 3):
            do_iter(k, k % 3, n)

        @pl.loop(3, 3 + nmain, step=3)
        def _(k):
            do_iter(4, 0, n, k_traced=k)
            do_iter(4, 1, n, k_traced=k + 1)
            do_iter(4, 2, n, k_traced=k + 2)

        for k in range(3 + nmain, n):
            do_iter(k, k % 3, n)
        s_wait((n - 1) % 3)

    @pl.when(s < 2)
    def _():
        pipeline(NCHT + 1)

    @pl.when(s >= 2)
    def _():
        pipeline(NCHT)

    plsc.subcore_barrier()

    pltpu.sync_copy(acc.at[pl.ds(abase, ACC_PER_TILE)],
                    out_hbm.at[c].at[pl.ds(abase, ACC_PER_TILE)])


@jax.jit
def _sc_scatter(feat, edge_index):
    mesh = plsc.VectorSubcoreMesh(core_axis_name="c", subcore_axis_name="s")
    k = pl.kernel(
        _sc_body,
        out_type=jax.ShapeDtypeStruct((NC, ACC_ROWS, D_C), jnp.float32),
        mesh=mesh,
        scratch_types=(
            [pltpu.VMEM((2, CH), jnp.int32) for _ in range(3)]
            + [pltpu.VMEM((CH, D_C), jnp.float32) for _ in range(3)]
            + [pltpu.VMEM_SHARED((ACC_ROWS, D_C), jnp.float32)]
            + [pltpu.SemaphoreType.DMA for _ in range(9)]
        ),
    )
    return k(feat, edge_index)


def _finish_body(p0_ref, p1_ref, f_ref, w_ref, o_ref):
    x = p0_ref[0] + p1_ref[0] + f_ref[...]
    y = lax.dot_general(x, w_ref[...], (((1,), (0,)), ((), ())),
                        preferred_element_type=jnp.float32,
                        precision=lax.Precision.DEFAULT)
    o_ref[...] = jnp.maximum(y, 0.0)


BR = 2000


@jax.jit
def _tc_finish(partials, feat, weight1):
    return pl.pallas_call(
        _finish_body,
        grid=(N_NODES_C // BR,),
        in_specs=[
            pl.BlockSpec((1, BR, D_C), lambda i: (0, i, 0)),
            pl.BlockSpec((1, BR, D_C), lambda i: (1, i, 0)),
            pl.BlockSpec((BR, D_C), lambda i: (i, 0)),
            pl.BlockSpec((D_C, D_C), lambda i: (0, 0)),
        ],
        out_specs=pl.BlockSpec((BR, D_C), lambda i: (i, 0)),
        out_shape=jax.ShapeDtypeStruct((N_NODES_C, D_C), jnp.float32),
    )(partials, partials, feat, weight1)


def kernel(feat, edge_index, weight1):
    ei = edge_index.astype(jnp.int32)
    partials = _sc_scatter(feat, ei)
    return _tc_finish(partials, feat, weight1)


# speedup vs baseline: 1.0013x; 1.0013x over previous
import jax
import jax.numpy as jnp
from jax import lax
from jax.experimental import pallas as pl
from jax.experimental.pallas import tpu as pltpu
from jax.experimental.pallas import tpu_sc as plsc

N_NODES_C = 10000
N_EDGES_C = 320000
D_C = 128

NC = 2
NS = 16
LANES = 16

CH = 128
NCHT = 78
NXTRA = 4
DUMMY = N_NODES_C
ACC_PER_TILE = 632
ACC_ROWS = NS * ACC_PER_TILE


def _sc_body(feat_hbm, ei_hbm, out_hbm, *refs):
    idx = refs[0:3]
    rows = refs[3:6]
    acc = refs[6]
    isems = refs[7:10]
    gsems = refs[10:13]
    ssems = refs[13:16]

    c = lax.axis_index("c")
    s = lax.axis_index("s")
    w = c * NS + s

    def idx_start(b, eoff):
        pltpu.async_copy(
            ei_hbm.at[pl.ds(0, 2), pl.ds(eoff, CH)], idx[b], isems[b])

    def idx_wait(b):
        pltpu.make_async_copy(
            ei_hbm.at[pl.ds(0, 2), pl.ds(0, CH)], idx[b], isems[b]).wait()

    def mask(b):
        for j in range(CH // LANES):
            sl = pl.ds(j * LANES, LANES)
            sv = idx[b][0, sl]
            dv = idx[b][1, sl]
            idx[b][1, sl] = jnp.where(
                sv == dv, jnp.full((LANES,), DUMMY, jnp.int32), dv)

    H = CH // 2

    def g_start(b):
        pltpu.async_copy(feat_hbm.at[idx[b].at[0, pl.ds(0, H)]],
                         rows[b].at[pl.ds(0, H)], gsems[b])
        pltpu.async_copy(feat_hbm.at[idx[b].at[0, pl.ds(H, H)]],
                         rows[b].at[pl.ds(H, H)], gsems[b])

    def g_wait(b):
        pltpu.make_async_copy(feat_hbm.at[idx[b].at[0, pl.ds(0, H)]],
                              rows[b].at[pl.ds(0, H)], gsems[b]).wait()
        pltpu.make_async_copy(feat_hbm.at[idx[b].at[0, pl.ds(H, H)]],
                              rows[b].at[pl.ds(H, H)], gsems[b]).wait()

    def s_start(b):
        pltpu.async_copy(rows[b], acc.at[idx[b].at[1]], ssems[b], add=True)

    def s_wait(b):
        pltpu.make_async_copy(rows[b], acc.at[idx[b].at[1]], ssems[b]).wait()

    @pl.loop(0, CH)
    def _(i):
        @pl.loop(0, D_C // LANES)
        def _(k):
            rows[0][i, pl.ds(k * LANES, LANES)] = jnp.zeros(
                (LANES,), jnp.float32)

    abase = s * ACC_PER_TILE

    @pl.loop(0, 4)
    def _(j):
        pltpu.sync_copy(rows[0], acc.at[pl.ds(abase + j * CH, CH)])
    pltpu.sync_copy(rows[0].at[pl.ds(0, ACC_PER_TILE - 4 * CH)],
                    acc.at[pl.ds(abase + 4 * CH, ACC_PER_TILE - 4 * CH)])

    plsc.subcore_barrier()

    ebase = w * (NCHT * CH)
    xoff = (NS * NC * NCHT + 2 * c + s) * CH

    def eoff_of(kk, static_k, n):
        if static_k == n - 1 and n == NCHT + 1:
            return xoff
        return ebase + kk * CH

    def do_iter(k, b, n, k_traced=None):
        kk = k if k_traced is None else k_traced
        b1, b2 = (b + 1) % 3, (b + 2) % 3
        if k + 1 < n:
            idx_wait(b1)
            mask(b1)
            g_start(b1)
        g_wait(b)
        s_start(b)
        if k >= 1:
            s_wait(b2)
        if k + 2 < n:
            idx_start(b2, eoff_of(kk + 2, k + 2, n))

    def pipeline(n):
        idx_start(0, eoff_of(0, 0, n))
        idx_start(1, eoff_of(1, 1, n))
        idx_wait(0)
        mask(0)
        g_start(0)

        nmain = ((n - 6) // 3) * 3

        for k in range(0, 3):
            do_iter(k, k % 3, n)

        @pl.loop(3, 3 + nmain, step=3)
        def _(k):
            do_iter(4, 0, n, k_traced=k)
            do_iter(4, 1, n, k_traced=k + 1)
            do_iter(4, 2, n, k_traced=k + 2)

        for k in range(3 + nmain, n):
            do_iter(k, k % 3, n)
        s_wait((n - 1) % 3)

    @pl.when(s < 2)
    def _():
        pipeline(NCHT + 1)

    @pl.when(s >= 2)
    def _():
        pipeline(NCHT)

    plsc.subcore_barrier()

    pltpu.sync_copy(acc.at[pl.ds(abase, ACC_PER_TILE)],
                    out_hbm.at[c].at[pl.ds(abase, ACC_PER_TILE)])


@jax.jit
def _sc_scatter(feat, edge_index):
    mesh = plsc.VectorSubcoreMesh(core_axis_name="c", subcore_axis_name="s")
    k = pl.kernel(
        _sc_body,
        out_type=jax.ShapeDtypeStruct((NC, ACC_ROWS, D_C), jnp.float32),
        mesh=mesh,
        scratch_types=(
            [pltpu.VMEM((2, CH), jnp.int32) for _ in range(3)]
            + [pltpu.VMEM((CH, D_C), jnp.float32) for _ in range(3)]
            + [pltpu.VMEM_SHARED((ACC_ROWS, D_C), jnp.float32)]
            + [pltpu.SemaphoreType.DMA for _ in range(9)]
        ),
    )
    return k(feat, edge_index)


def _finish_body(p0_ref, p1_ref, f_ref, w_ref, o_ref):
    x = p0_ref[0] + p1_ref[0] + f_ref[...]
    y = lax.dot_general(x, w_ref[...], (((1,), (0,)), ((), ())),
                        preferred_element_type=jnp.float32,
                        precision=lax.Precision.DEFAULT)
    o_ref[...] = jnp.maximum(y, 0.0)


BR = 2000


@jax.jit
def _tc_finish(partials, feat, weight1):
    return pl.pallas_call(
        _finish_body,
        grid=(N_NODES_C // BR,),
        in_specs=[
            pl.BlockSpec((1, BR, D_C), lambda i: (0, i, 0)),
            pl.BlockSpec((1, BR, D_C), lambda i: (1, i, 0)),
            pl.BlockSpec((BR, D_C), lambda i: (i, 0)),
            pl.BlockSpec((D_C, D_C), lambda i: (0, 0)),
        ],
        out_specs=pl.BlockSpec((BR, D_C), lambda i: (i, 0)),
        out_shape=jax.ShapeDtypeStruct((N_NODES_C, D_C), jnp.float32),
    )(partials, partials, feat, weight1)


def kernel(feat, edge_index, weight1):
    ei = edge_index.astype(jnp.int32)
    partials = _sc_scatter(feat, ei)
    return _tc_finish(partials, feat, weight1)
